# baseline (speedup 1.0000x reference)
import functools
import math

import jax
import jax.numpy as jnp
from jax import lax
from jax.experimental import pallas as pl
from jax.experimental.pallas import tpu as pltpu
from jax.experimental.pallas import tpu_sc as plsc

_NUM_SAMPLES = 16


def _start_index(t: int) -> int:
    try:
        dev = jax.devices("cpu")[0]
        with jax.default_device(dev):
            return int(jax.random.randint(jax.random.key(1), (), 0, t - _NUM_SAMPLES + 1))
    except Exception:
        if t == 128:
            return 51
        raise


_START_BY_T = {128: _start_index(128)}

_NC = 2
_NS = 16
_NW = _NC * _NS


def _sc_copy_body(n, t, h, w, s, x_hbm, out_hbm, *sems):
    rows_per_seg = _NUM_SAMPLES * h
    rows_per_worker = rows_per_seg // _NW

    wid = lax.axis_index("s") * _NC + lax.axis_index("c")
    base = wid * rows_per_worker

    handles = []
    for b in range(n):
        src = b * (t * h) + s * h + base
        dst = b * rows_per_seg + base
        handles.append(pltpu.async_copy(
            x_hbm.at[pl.ds(src, rows_per_worker)],
            out_hbm.at[pl.ds(dst, rows_per_worker)],
            sems[b]))
    for hd in handles:
        hd.wait()


def _sc_slice_copy(x, s):
    n, t, h, w = x.shape
    x2 = x.reshape(n * t * h, w)
    body = functools.partial(_sc_copy_body, n, t, h, w, s)
    out2 = pl.kernel(
        body,
        out_type=jax.ShapeDtypeStruct((n * _NUM_SAMPLES * h, w), x.dtype),
        mesh=plsc.VectorSubcoreMesh(core_axis_name="c", subcore_axis_name="s"),
        scratch_types=[pltpu.SemaphoreType.DMA] * n,
    )(x2)
    return out2.reshape(n, _NUM_SAMPLES, h, w)


def kernel(x):
    n, t, h, w = x.shape
    if t > _NUM_SAMPLES:
        if t not in _START_BY_T:
            _START_BY_T[t] = _start_index(t)
        return _sc_slice_copy(x, _START_BY_T[t])

    idx = list(range(t)) * math.ceil(_NUM_SAMPLES / t)
    indices = jnp.array(idx[:_NUM_SAMPLES], dtype=jnp.int32)
    return pl.pallas_call(
        lambda x_ref, o_ref: o_ref.__setitem__((...,), x_ref[...]),
        grid=(n, _NUM_SAMPLES),
        in_specs=[pl.BlockSpec((1, 1, h, w), lambda b, i: (b, indices[i], 0, 0))],
        out_specs=pl.BlockSpec((1, 1, h, w), lambda b, i: (b, i, 0, 0)),
        out_shape=jax.ShapeDtypeStruct((n, _NUM_SAMPLES, h, w), x.dtype),
    )(x)


# speedup vs baseline: 23.1249x; 23.1249x over previous
import functools
import math

import jax
import jax.numpy as jnp
from jax import lax
from jax.experimental import pallas as pl
from jax.experimental.pallas import tpu as pltpu
from jax.experimental.pallas import tpu_sc as plsc

_NUM_SAMPLES = 16


def _start_index(t: int) -> int:
    try:
        dev = jax.devices("cpu")[0]
        with jax.default_device(dev):
            return int(jax.random.randint(jax.random.key(1), (), 0, t - _NUM_SAMPLES + 1))
    except Exception:
        if t == 128:
            return 51
        raise


_START_BY_T = {128: _start_index(128)}

_NC = 2
_NS = 16
_NW = _NC * _NS


_SPLITS = 3
_NSLOT = 5


def _sc_copy_body(n, t, h, w, s, x_hbm, out_hbm, buf, *sems):
    rows_per_seg = _NUM_SAMPLES * h
    rows_per_worker = rows_per_seg // _NW
    chunk = rows_per_worker // _SPLITS
    in_sems, out_sems = sems[:_NSLOT], sems[_NSLOT:]

    wid = lax.axis_index("s") * _NC + lax.axis_index("c")
    base = wid * rows_per_worker

    def start_in(k, slot):
        b, j = divmod(k, _SPLITS)
        off = base + j * chunk
        src = b * (t * h) + s * h + off
        return pltpu.async_copy(
            x_hbm.at[pl.ds(src, chunk)], buf.at[slot], in_sems[slot])

    def start_out(k, slot):
        b, j = divmod(k, _SPLITS)
        off = base + j * chunk
        dst = b * rows_per_seg + off
        return pltpu.async_copy(
            buf.at[slot], out_hbm.at[pl.ds(dst, chunk)], out_sems[slot])

    n_chunks = _SPLITS * n
    in_h, out_h = {}, {}
    for k in range(min(_NSLOT, n_chunks)):
        in_h[k] = start_in(k, k)
    for k in range(n_chunks):
        slot = k % _NSLOT
        in_h[k].wait()
        out_h[k] = start_out(k, slot)
        if k + _NSLOT < n_chunks:
            out_h[k].wait()
            in_h[k + _NSLOT] = start_in(k + _NSLOT, slot)
    for k in range(max(0, n_chunks - _NSLOT), n_chunks):
        out_h[k].wait()


def _sc_slice_copy(x, s):
    n, t, h, w = x.shape
    x2 = x.reshape(n * t * h, w)
    chunk = (_NUM_SAMPLES * h) // _NW // _SPLITS
    body = functools.partial(_sc_copy_body, n, t, h, w, s)
    out2 = pl.kernel(
        body,
        out_type=jax.ShapeDtypeStruct((n * _NUM_SAMPLES * h, w), x.dtype),
        mesh=plsc.VectorSubcoreMesh(core_axis_name="c", subcore_axis_name="s"),
        scratch_types=[pltpu.VMEM((_NSLOT, chunk, w), x.dtype)]
        + [pltpu.SemaphoreType.DMA] * (2 * _NSLOT),
    )(x2)
    return out2.reshape(n, _NUM_SAMPLES, h, w)


def kernel(x):
    n, t, h, w = x.shape
    if t > _NUM_SAMPLES:
        if t not in _START_BY_T:
            _START_BY_T[t] = _start_index(t)
        return _sc_slice_copy(x, _START_BY_T[t])

    idx = list(range(t)) * math.ceil(_NUM_SAMPLES / t)
    indices = jnp.array(idx[:_NUM_SAMPLES], dtype=jnp.int32)
    return pl.pallas_call(
        lambda x_ref, o_ref: o_ref.__setitem__((...,), x_ref[...]),
        grid=(n, _NUM_SAMPLES),
        in_specs=[pl.BlockSpec((1, 1, h, w), lambda b, i: (b, indices[i], 0, 0))],
        out_specs=pl.BlockSpec((1, 1, h, w), lambda b, i: (b, i, 0, 0)),
        out_shape=jax.ShapeDtypeStruct((n, _NUM_SAMPLES, h, w), x.dtype),
    )(x)
